# baseline (speedup 1.0000x reference)
import math
import functools

import numpy as np
import jax
import jax.numpy as jnp
from jax import lax
from jax.experimental import pallas as pl
from jax.experimental.pallas import tpu as pltpu
from jax.experimental.pallas import tpu_sc as plsc

D_MODEL = 1024
MAX_LEN = 8192
BATCH = 4
VOCAB = 258
NB_SC = 2
NB_TC = BATCH - NB_SC
LANES = 16
NUM_CORES = 2
NUM_SUBCORES = 16
NUM_WORKERS = NUM_CORES * NUM_SUBCORES
SEQ_PER_WORKER = MAX_LEN // NUM_WORKERS
CHUNK = 16
CHUNKS_PER_WORKER = SEQ_PER_WORKER // CHUNK
STEPS = CHUNKS_PER_WORKER * NB_SC
ROWS_PER_BATCH = MAX_LEN // CHUNK
NBUF = 5
TC_BLK = 1024


def _make_pe(max_len, d_model):
    pos = np.arange(max_len, dtype=np.float32)[:, None]
    div = np.exp(np.arange(0, d_model, 2, dtype=np.float32)
                 * (-math.log(10000.0) / d_model))
    pe = np.zeros((max_len, d_model), dtype=np.float32)
    pe[:, 0::2] = np.sin(pos * div)
    pe[:, 1::2] = np.cos(pos * div)
    return jnp.asarray(pe)


_mesh = plsc.VectorSubcoreMesh(
    core_axis_name="c", subcore_axis_name="s",
    num_cores=NUM_CORES, num_subcores=NUM_SUBCORES)


@functools.partial(
    pl.kernel,
    out_type=jax.ShapeDtypeStruct((NB_SC * MAX_LEN, D_MODEL), jnp.float32),
    mesh=_mesh,
    scratch_types=[
        pltpu.VMEM((STEPS, CHUNK), jnp.int32),
        pltpu.VMEM((NBUF, CHUNK, D_MODEL), jnp.float32),
        pltpu.VMEM((2, CHUNK, D_MODEL), jnp.float32),
        pltpu.SemaphoreType.DMA((NBUF,)),
        pltpu.SemaphoreType.DMA((NBUF,)),
        pltpu.SemaphoreType.DMA((2,)),
    ],
)
def _sc_embed(x_hbm, table_hbm, pe_hbm, out_hbm,
              idx_all, res_v, pe_v, gsem, ssem, psem):
    wid = lax.axis_index("s") * NUM_CORES + lax.axis_index("c")
    s_base = pl.multiple_of(wid * SEQ_PER_WORKER, SEQ_PER_WORKER)
    row_base = pl.multiple_of(wid * CHUNKS_PER_WORKER, CHUNKS_PER_WORKER)

    for b in range(NB_SC):
        pltpu.sync_copy(
            x_hbm.at[pl.ds(b * ROWS_PER_BATCH + row_base, CHUNKS_PER_WORKER)],
            idx_all.at[pl.ds(b * CHUNKS_PER_WORKER, CHUNKS_PER_WORKER)])

    def gather_copy(i, slot):
        b = lax.rem(i, NB_SC)
        j = lax.div(i, NB_SC)
        return pltpu.make_async_copy(
            table_hbm.at[idx_all.at[b * CHUNKS_PER_WORKER + j]],
            res_v.at[slot], gsem.at[slot])

    def store_copy(i, slot):
        b = lax.rem(i, NB_SC)
        j = lax.div(i, NB_SC)
        off = pl.multiple_of(b * MAX_LEN + s_base + j * CHUNK, CHUNK)
        return pltpu.make_async_copy(
            res_v.at[slot], out_hbm.at[pl.ds(off, CHUNK)], ssem.at[slot])

    def pe_copy(j, pslot):
        return pltpu.make_async_copy(
            pe_hbm.at[pl.ds(pl.multiple_of(s_base + j * CHUNK, CHUNK), CHUNK)],
            pe_v.at[pslot], psem.at[pslot])

    pe_copy(0, 0).start()
    pe_copy(1, 1).start()
    gather_copy(0, 0).start()
    gather_copy(1, 1).start()
    gather_copy(2, 2).start()

    def group(g, carry):
        pj = lax.rem(g, 2)
        pe_copy(g, pj).wait()
        for k in range(NB_SC):
            i = g * NB_SC + k
            slot = i % NBUF
            pslot = (i + 3) % NBUF
            @pl.when(i + 3 < STEPS)
            def _():
                pl.when(i >= 2)(lambda: store_copy(i - 2, pslot).wait())
                gather_copy(i + 3, pslot).start()

            gather_copy(i, slot).wait()

            @plsc.parallel_loop(0, CHUNK, 1)
            def row_add(r):
                for kc in range(D_MODEL // LANES):
                    sl = pl.ds(kc * LANES, LANES)
                    plsc.addupdate(res_v.at[slot, r, sl], pe_v[pj, r, sl])

            store_copy(i, slot).start()
        pl.when(g + 2 < CHUNKS_PER_WORKER)(lambda: pe_copy(g + 2, pj).start())
        return carry

    lax.fori_loop(0, CHUNKS_PER_WORKER, group, 0, unroll=False)

    for t in range(NBUF):
        store_copy(STEPS - NBUF + t, (STEPS - NBUF + t) % NBUF).wait()


def _tc_body(x_ref, table_ref, pe_ref, out_ref):
    ids = x_ref[...]
    oh = (ids[:, None] ==
          lax.broadcasted_iota(jnp.int32, (TC_BLK, VOCAB), 1)
          ).astype(jnp.float32)
    emb = jnp.dot(oh, table_ref[...], preferred_element_type=jnp.float32)
    out_ref[...] = emb + pe_ref[...]


def _tc_embed(x_tc, table, pe):
    n_blk = (NB_TC * MAX_LEN) // TC_BLK
    return pl.pallas_call(
        _tc_body,
        grid=(n_blk,),
        in_specs=[
            pl.BlockSpec((TC_BLK,), lambda i: (i,)),
            pl.BlockSpec((VOCAB, D_MODEL), lambda i: (0, 0)),
            pl.BlockSpec((TC_BLK, D_MODEL),
                         lambda i: (i % (MAX_LEN // TC_BLK), 0)),
        ],
        out_specs=pl.BlockSpec((TC_BLK, D_MODEL), lambda i: (i, 0)),
        out_shape=jax.ShapeDtypeStruct((NB_TC * MAX_LEN, D_MODEL),
                                       jnp.float32),
    )(x_tc, table, pe)


def kernel(x, table):
    pe = _make_pe(MAX_LEN, D_MODEL)
    x = x.astype(jnp.int32)
    idx_sc = x[:NB_SC].reshape(NB_SC * ROWS_PER_BATCH, CHUNK)
    sc_out = _sc_embed(idx_sc, table, pe)
    tc_out = _tc_embed(x[NB_SC:].reshape(NB_TC * MAX_LEN), table, pe)
    out = jnp.concatenate(
        [sc_out.reshape(NB_SC, MAX_LEN, D_MODEL),
         tc_out.reshape(NB_TC, MAX_LEN, D_MODEL)], axis=0)
    return out


# speedup vs baseline: 1.2040x; 1.2040x over previous
import math
import functools

import numpy as np
import jax
import jax.numpy as jnp
from jax import lax
from jax.experimental import pallas as pl
from jax.experimental.pallas import tpu as pltpu
from jax.experimental.pallas import tpu_sc as plsc

D_MODEL = 1024
MAX_LEN = 8192
BATCH = 4
VOCAB = 258
NB_SC = 2
NB_TC = BATCH - NB_SC
LANES = 16
NUM_CORES = 2
NUM_SUBCORES = 16
NUM_WORKERS = NUM_CORES * NUM_SUBCORES
SEQ_PER_WORKER = MAX_LEN // NUM_WORKERS
CHUNK = 16
CHUNKS_PER_WORKER = SEQ_PER_WORKER // CHUNK
STEPS = CHUNKS_PER_WORKER * NB_SC
ROWS_PER_BATCH = MAX_LEN // CHUNK
NBUF = 5
TC_BLK = 1024


def _make_pe(max_len, d_model):
    pos = np.arange(max_len, dtype=np.float32)[:, None]
    div = np.exp(np.arange(0, d_model, 2, dtype=np.float32)
                 * (-math.log(10000.0) / d_model))
    pe = np.zeros((max_len, d_model), dtype=np.float32)
    pe[:, 0::2] = np.sin(pos * div)
    pe[:, 1::2] = np.cos(pos * div)
    return pe


_mesh = plsc.VectorSubcoreMesh(
    core_axis_name="c", subcore_axis_name="s",
    num_cores=NUM_CORES, num_subcores=NUM_SUBCORES)


PE_ROWS = CHUNK * 8


@functools.partial(
    pl.kernel,
    out_type=jax.ShapeDtypeStruct((BATCH * MAX_LEN, D_MODEL), jnp.float32),
    mesh=_mesh,
    scratch_types=[
        pltpu.VMEM((STEPS, CHUNK), jnp.int32),
        pltpu.VMEM((NBUF, CHUNK, D_MODEL), jnp.float32),
        pltpu.VMEM((2, PE_ROWS, 128), jnp.float32),
        pltpu.SemaphoreType.DMA((NBUF,)),
        pltpu.SemaphoreType.DMA((NBUF,)),
        pltpu.SemaphoreType.DMA((2,)),
    ],
)
def _sc_embed(x_hbm, table_hbm, pe_hbm, out_hbm,
              idx_all, res_v, pe_v, gsem, ssem, psem):
    wid = lax.axis_index("s") * NUM_CORES + lax.axis_index("c")
    s_base = pl.multiple_of(wid * SEQ_PER_WORKER, SEQ_PER_WORKER)
    row_base = pl.multiple_of(wid * CHUNKS_PER_WORKER, CHUNKS_PER_WORKER)

    for b in range(NB_SC):
        pltpu.sync_copy(
            x_hbm.at[pl.ds(b * ROWS_PER_BATCH + row_base, CHUNKS_PER_WORKER)],
            idx_all.at[pl.ds(b * CHUNKS_PER_WORKER, CHUNKS_PER_WORKER)])

    def gather_copy(i, slot):
        b = lax.rem(i, NB_SC)
        j = lax.div(i, NB_SC)
        return pltpu.make_async_copy(
            table_hbm.at[idx_all.at[b * CHUNKS_PER_WORKER + j]],
            res_v.at[slot], gsem.at[slot])

    def store_copy(i, slot):
        b = lax.rem(i, NB_SC)
        j = lax.div(i, NB_SC)
        off = pl.multiple_of(b * MAX_LEN + s_base + j * CHUNK, CHUNK)
        return pltpu.make_async_copy(
            res_v.at[slot], out_hbm.at[pl.ds(off, CHUNK)], ssem.at[slot])

    def pe_copy(j, pslot):
        r0 = pl.multiple_of((s_base + j * CHUNK) * 8, PE_ROWS)
        return pltpu.make_async_copy(
            pe_hbm.at[pl.ds(r0, PE_ROWS)], pe_v.at[pslot], psem.at[pslot])

    pe_copy(0, 0).start()
    pe_copy(1, 1).start()
    gather_copy(0, 0).start()
    gather_copy(1, 1).start()
    gather_copy(2, 2).start()

    def group(g, carry):
        pj = lax.rem(g, 2)
        pe_copy(g, pj).wait()
        for k in range(NB_SC):
            i = g * NB_SC + k
            slot = i % NBUF
            pslot = (i + 3) % NBUF
            @pl.when(i + 3 < STEPS)
            def _():
                pl.when(i >= 2)(lambda: store_copy(i - 2, pslot).wait())
                gather_copy(i + 3, pslot).start()

            gather_copy(i, slot).wait()

            @plsc.parallel_loop(0, CHUNK, 1)
            def row_add(r):
                for kc in range(D_MODEL // LANES):
                    sl = pl.ds(kc * LANES, LANES)
                    psl = pl.ds((kc % 8) * LANES, LANES)
                    plsc.addupdate(res_v.at[slot, r, sl],
                                   pe_v[pj, r * 8 + kc // 8, psl])

            store_copy(i, slot).start()
        pl.when(g + 2 < CHUNKS_PER_WORKER)(lambda: pe_copy(g + 2, pj).start())
        return carry

    lax.fori_loop(0, CHUNKS_PER_WORKER, group, 0, unroll=False)

    for t in range(NBUF):
        store_copy(STEPS - NBUF + t, (STEPS - NBUF + t) % NBUF).wait()


def _tc_body(x_ref, hi_ref, lo_ref, pe_ref, out_ref):
    ids = x_ref[...]
    oh = (ids[:, None] ==
          lax.broadcasted_iota(jnp.int32, (TC_BLK, VOCAB), 1)
          ).astype(jnp.bfloat16)
    emb = (jnp.dot(oh, hi_ref[...], preferred_element_type=jnp.float32)
           + jnp.dot(oh, lo_ref[...], preferred_element_type=jnp.float32))
    out_ref[...] = emb + pe_ref[...]


def _tc_embed(x_tc, table_hi, table_lo, pe):
    n_blk = (NB_TC * MAX_LEN) // TC_BLK
    return pl.pallas_call(
        _tc_body,
        grid=(n_blk,),
        in_specs=[
            pl.BlockSpec((TC_BLK,), lambda i: (i,)),
            pl.BlockSpec((VOCAB, D_MODEL), lambda i: (0, 0)),
            pl.BlockSpec((VOCAB, D_MODEL), lambda i: (0, 0)),
            pl.BlockSpec((TC_BLK, D_MODEL),
                         lambda i: (i % (MAX_LEN // TC_BLK), 0)),
        ],
        out_specs=pl.BlockSpec((TC_BLK, D_MODEL), lambda i: (i, 0)),
        out_shape=jax.ShapeDtypeStruct((NB_TC * MAX_LEN, D_MODEL),
                                       jnp.float32),
    )(x_tc, table_hi, table_lo, pe)


def kernel(x, table):
    pe = _make_pe(MAX_LEN, D_MODEL)
    pe_sc = jnp.asarray(pe.reshape(MAX_LEN * 8, 128))
    pe_tc = jnp.asarray(pe)
    x = x.astype(jnp.int32)
    idx_sc = x[:NB_SC].reshape(NB_SC * ROWS_PER_BATCH, CHUNK)
    sc_out = _sc_embed(idx_sc, table, pe_sc)
    table_hi = table.astype(jnp.bfloat16)
    table_lo = (table - table_hi.astype(jnp.float32)).astype(jnp.bfloat16)
    tc_out = _tc_embed(x[NB_SC:].reshape(NB_TC * MAX_LEN),
                       table_hi, table_lo, pe_tc)
    out = lax.dynamic_update_slice(sc_out, tc_out, (NB_SC * MAX_LEN, 0))
    return out.reshape(BATCH, MAX_LEN, D_MODEL)


# speedup vs baseline: 1.3897x; 1.1542x over previous
import math
import functools

import numpy as np
import jax
import jax.numpy as jnp
from jax import lax
from jax.experimental import pallas as pl
from jax.experimental.pallas import tpu as pltpu
from jax.experimental.pallas import tpu_sc as plsc

D_MODEL = 1024
MAX_LEN = 8192
BATCH = 4
LANES = 16
NUM_CORES = 2
NUM_SUBCORES = 16
NUM_WORKERS = NUM_CORES * NUM_SUBCORES
SEQ_PER_WORKER = MAX_LEN // NUM_WORKERS
CHUNK = 16
CHUNKS_PER_WORKER = SEQ_PER_WORKER // CHUNK
STEPS = CHUNKS_PER_WORKER * BATCH
ROWS_PER_BATCH = MAX_LEN // CHUNK
NQ = MAX_LEN // CHUNK
NBUF = 3


def _make_pe_factors():
    w = np.exp(np.arange(0, D_MODEL, 2, dtype=np.float64)
               * (-math.log(10000.0) / D_MODEL))
    th = (CHUNK * np.arange(NQ, dtype=np.float64))[:, None] * w[None, :]
    ph = np.arange(CHUNK, dtype=np.float64)[:, None] * w[None, :]
    a = np.zeros((NQ, D_MODEL), np.float32)
    b = np.zeros((NQ, D_MODEL), np.float32)
    c = np.zeros((CHUNK, D_MODEL), np.float32)
    s = np.zeros((CHUNK, D_MODEL), np.float32)
    a[:, 0::2], a[:, 1::2] = np.sin(th), np.cos(th)
    b[:, 0::2], b[:, 1::2] = np.cos(th), -np.sin(th)
    c[:, 0::2], c[:, 1::2] = np.cos(ph), np.cos(ph)
    s[:, 0::2], s[:, 1::2] = np.sin(ph), np.sin(ph)
    return a, b, c, s


_mesh = plsc.VectorSubcoreMesh(
    core_axis_name="c", subcore_axis_name="s",
    num_cores=NUM_CORES, num_subcores=NUM_SUBCORES)


@functools.partial(
    pl.kernel,
    out_type=jax.ShapeDtypeStruct((BATCH * MAX_LEN, D_MODEL), jnp.float32),
    mesh=_mesh,
    scratch_types=[
        pltpu.VMEM((STEPS, CHUNK), jnp.int32),
        pltpu.VMEM((NBUF, CHUNK, D_MODEL), jnp.float32),
        pltpu.VMEM((2, CHUNKS_PER_WORKER, D_MODEL), jnp.float32),
        pltpu.VMEM((2, CHUNK, D_MODEL), jnp.float32),
        pltpu.SemaphoreType.DMA((NBUF,)),
        pltpu.SemaphoreType.DMA((NBUF,)),
    ],
)
def _sc_embed(x_hbm, table_hbm, a_hbm, b_hbm, c_hbm, s_hbm, out_hbm,
              idx_all, res_v, ab_v, cs_v, gsem, ssem):
    wid = lax.axis_index("s") * NUM_CORES + lax.axis_index("c")
    s_base = pl.multiple_of(wid * SEQ_PER_WORKER, SEQ_PER_WORKER)
    row_base = pl.multiple_of(wid * CHUNKS_PER_WORKER, CHUNKS_PER_WORKER)

    for b in range(BATCH):
        pltpu.sync_copy(
            x_hbm.at[pl.ds(b * ROWS_PER_BATCH + row_base, CHUNKS_PER_WORKER)],
            idx_all.at[pl.ds(b * CHUNKS_PER_WORKER, CHUNKS_PER_WORKER)])
    pltpu.sync_copy(a_hbm.at[pl.ds(row_base, CHUNKS_PER_WORKER)], ab_v.at[0])
    pltpu.sync_copy(b_hbm.at[pl.ds(row_base, CHUNKS_PER_WORKER)], ab_v.at[1])
    pltpu.sync_copy(c_hbm, cs_v.at[0])
    pltpu.sync_copy(s_hbm, cs_v.at[1])

    def gather_copy(i, slot):
        b = lax.rem(i, BATCH)
        j = lax.div(i, BATCH)
        return pltpu.make_async_copy(
            table_hbm.at[idx_all.at[b * CHUNKS_PER_WORKER + j]],
            res_v.at[slot], gsem.at[slot])

    def store_copy(i, slot):
        b = lax.rem(i, BATCH)
        j = lax.div(i, BATCH)
        off = pl.multiple_of(b * MAX_LEN + s_base + j * CHUNK, CHUNK)
        return pltpu.make_async_copy(
            res_v.at[slot], out_hbm.at[pl.ds(off, CHUNK)], ssem.at[slot])

    gather_copy(0, 0).start()

    def step_fn(i, carry):
        slot = lax.rem(i, NBUF)
        nslot = lax.rem(i + 1, NBUF)
        j = lax.div(i, BATCH)

        @pl.when(i + 1 < STEPS)
        def _():
            pl.when(i >= 2)(lambda: store_copy(i - 2, nslot).wait())
            gather_copy(i + 1, nslot).start()

        gather_copy(i, slot).wait()

        @plsc.parallel_loop(0, D_MODEL // LANES, 1)
        def col_add(kc):
            sl = pl.ds(kc * LANES, LANES)
            a = ab_v[0, j, sl]
            b = ab_v[1, j, sl]
            for r in range(CHUNK):
                pe = a * cs_v[0, r, sl] + b * cs_v[1, r, sl]
                plsc.addupdate(res_v.at[slot, r, sl], pe)

        store_copy(i, slot).start()
        return carry

    lax.fori_loop(0, STEPS, step_fn, 0, unroll=False)

    for t in range(STEPS - NBUF, STEPS):
        store_copy(t, t % NBUF).wait()


def kernel(x, table):
    a, b, c, s = _make_pe_factors()
    idx = x.reshape(BATCH * ROWS_PER_BATCH, CHUNK).astype(jnp.int32)
    out = _sc_embed(idx, table, jnp.asarray(a), jnp.asarray(b),
                    jnp.asarray(c), jnp.asarray(s))
    return out.reshape(BATCH, MAX_LEN, D_MODEL)
